# baseline (speedup 1.0000x reference)
import functools

import jax
import jax.numpy as jnp
from jax import lax
from jax.experimental import pallas as pl
from jax.experimental.pallas import tpu as pltpu
from jax.experimental.pallas import tpu_sc as plsc

TOKENS = 8192
HIDDEN = 2048
EXPERTS = 64
TOPK = 8
SCALE = 2.5

BLK = 512

_INFO = plsc.get_sparse_core_info()
NC = _INFO.num_cores
NS = _INFO.num_subcores
NW = NC * NS
TPW = TOKENS // NW
NGRP = TPW // 16


def _softmax_body(h_ref, w_ref, b_ref, p_ref):
    h = h_ref[...]
    w = w_ref[...]
    logits = jnp.dot(h, w, preferred_element_type=jnp.float32)
    m = jnp.max(logits, axis=-1, keepdims=True)
    e = jnp.exp(logits - m)
    s = jnp.sum(e, axis=-1, keepdims=True)
    p_ref[...] = e / s + b_ref[...]


def _tc_biased(hidden_states, wt, bias):
    return pl.pallas_call(
        _softmax_body,
        grid=(TOKENS // BLK,),
        in_specs=[
            pl.BlockSpec((BLK, HIDDEN), lambda i: (i, 0)),
            pl.BlockSpec((HIDDEN, EXPERTS), lambda i: (0, 0)),
            pl.BlockSpec((1, EXPERTS), lambda i: (0, 0)),
        ],
        out_specs=pl.BlockSpec((BLK, EXPERTS), lambda i: (i, 0)),
        out_shape=jax.ShapeDtypeStruct((TOKENS, EXPERTS), jnp.float32),
    )(hidden_states, wt, bias)


def _sc_topk_body(p_hbm, b_hbm, w_hbm, i_hbm, p_v, b_v, ow_v, oi_v):
    c = lax.axis_index("c")
    s = lax.axis_index("s")
    wid = c * NS + s
    base = wid * (TPW * EXPERTS)

    pltpu.sync_copy(p_hbm.at[pl.ds(base, TPW * EXPERTS)], p_v)
    pltpu.sync_copy(b_hbm, b_v)

    iota = lax.iota(jnp.int32, 16)
    iota_e = iota * EXPERTS
    iota_k = iota * TOPK

    def group(g, _):
        fbase = iota_e + g * (16 * EXPERTS)
        vals = [jnp.full((16,), -1e30, jnp.float32) for _ in range(TOPK)]
        idxs = [jnp.zeros((16,), jnp.int32) for _ in range(TOPK)]
        for e in range(EXPERTS):
            sc = plsc.load_gather(p_v, [fbase + e])
            si = jnp.full((16,), e, jnp.int32)
            for j in range(TOPK):
                gt = sc > vals[j]
                nv = jnp.maximum(sc, vals[j])
                sc = jnp.minimum(sc, vals[j])
                ni = jnp.where(gt, si, idxs[j])
                si = jnp.where(gt, idxs[j], si)
                vals[j] = nv
                idxs[j] = ni
        wbase = iota_k + g * (16 * TOPK)
        for j in range(TOPK):
            bj = plsc.load_gather(b_v, [idxs[j]])
            wj = (vals[j] - bj) * SCALE
            plsc.store_scatter(ow_v, [wbase + j], wj)
            plsc.store_scatter(oi_v, [wbase + j], idxs[j])
        return 0

    lax.fori_loop(0, NGRP, group, 0)

    obase = wid * (TPW * TOPK)
    pltpu.sync_copy(ow_v, w_hbm.at[pl.ds(obase, TPW * TOPK)])
    pltpu.sync_copy(oi_v, i_hbm.at[pl.ds(obase, TPW * TOPK)])


_sc_topk = pl.kernel(
    _sc_topk_body,
    out_type=[
        jax.ShapeDtypeStruct((TOKENS * TOPK,), jnp.float32),
        jax.ShapeDtypeStruct((TOKENS * TOPK,), jnp.int32),
    ],
    mesh=plsc.VectorSubcoreMesh(core_axis_name="c", subcore_axis_name="s"),
    compiler_params=pltpu.CompilerParams(needs_layout_passes=False),
    scratch_types=[
        pltpu.VMEM((TPW * EXPERTS,), jnp.float32),
        pltpu.VMEM((EXPERTS,), jnp.float32),
        pltpu.VMEM((TPW * TOPK,), jnp.float32),
        pltpu.VMEM((TPW * TOPK,), jnp.int32),
    ],
)


@jax.jit
def kernel(hidden_states, classifier_weight, e_score_correction_bias):
    wt = classifier_weight.T
    bias = e_score_correction_bias.reshape(1, EXPERTS)
    biased = _tc_biased(hidden_states, wt, bias)
    w_flat, i_flat = _sc_topk(biased.reshape(-1), e_score_correction_bias)
    return w_flat.reshape(TOKENS, TOPK), i_flat.reshape(TOKENS, TOPK)
